# baseline (speedup 1.0000x reference)
import functools

import jax
import jax.numpy as jnp
from jax import lax
from jax.experimental import pallas as pl
from jax.experimental.pallas import tpu as pltpu
from jax.experimental.pallas import tpu_sc as plsc

_L = 16
_CHUNK = 18432


def _pwlu_sc_body(n_slab_per_w, slab_len, n_ch, x_hbm, params_hbm, out_hbm,
                  params_v, fp_ref, sl_ref, xbuf):
    wid = lax.axis_index("s") * 2 + lax.axis_index("c")

    iota = lax.iota(jnp.int32, _L)
    zeros_i = jnp.zeros((_L,), jnp.int32)
    n_chunks = slab_len // _CHUNK
    n_vecs = _CHUNK // _L

    for j in range(n_slab_per_w):
        slab = wid * n_slab_per_w + j
        c = lax.rem(slab, n_ch)
        pltpu.sync_copy(params_hbm.at[pl.ds(c * _L, _L)], params_v)

        def bcast(off):
            return plsc.load_gather(params_v, [zeros_i + off])

        lb = bcast(7)
        rb = bcast(8)
        ls = bcast(9)
        rs = bcast(10)
        p0 = bcast(0)
        rl = rb - lb
        inv = 1.0 / rl
        sim_left = lb - rl

        g_lo = plsc.load_gather(params_v, [jnp.maximum(iota - 1, 0)])
        fp_vec = jnp.where(iota == 0, p0 - ls * rl, g_lo)
        g_hi = plsc.load_gather(params_v, [jnp.minimum(iota, 6)])
        s_int = (g_hi - g_lo) * inv
        sl_vec = jnp.where(iota == 0, ls,
                           jnp.where(iota >= 7, rs, s_int))
        fp_ref[...] = fp_vec
        sl_ref[...] = sl_vec

        for k in range(n_chunks):
            base = slab * slab_len + k * _CHUNK
            pltpu.sync_copy(x_hbm.at[pl.ds(base, _CHUNK)], xbuf)

            def vec_body(i, carry):
                o = i * _L
                xv = xbuf[pl.ds(o, _L)]
                t = (xv - sim_left) * inv
                tc = jnp.minimum(jnp.maximum(t, 0.0), 7.007)
                ri = tc.astype(jnp.int32)
                d = t - ri.astype(jnp.float32)
                fpv = plsc.load_gather(fp_ref, [ri])
                slv = plsc.load_gather(sl_ref, [ri])
                xbuf[pl.ds(o, _L)] = fpv + d * slv
                return carry

            lax.fori_loop(0, n_vecs, vec_body, 0)
            pltpu.sync_copy(xbuf, out_hbm.at[pl.ds(base, _CHUNK)])


def kernel(x, points, bounds, left_slopes, right_slopes):
    B, C, H, W = x.shape
    n = B * C * H * W
    slab_len = H * W
    n_slabs = B * C
    n_workers = 32
    assert n_slabs % n_workers == 0 and slab_len % _CHUNK == 0

    params = jnp.concatenate(
        [points, bounds, left_slopes[:, None], right_slopes[:, None],
         jnp.zeros((C, _L - 11), jnp.float32)], axis=1).reshape(-1)

    mesh = plsc.VectorSubcoreMesh(core_axis_name="c", subcore_axis_name="s")
    body = functools.partial(
        _pwlu_sc_body, n_slabs // n_workers, slab_len, C)
    run = pl.kernel(
        body,
        mesh=mesh,
        out_type=jax.ShapeDtypeStruct((n,), jnp.float32),
        scratch_types=[
            pltpu.VMEM((_L,), jnp.float32),
            pltpu.VMEM((_L,), jnp.float32),
            pltpu.VMEM((_L,), jnp.float32),
            pltpu.VMEM((_CHUNK,), jnp.float32),
        ],
        compiler_params=pltpu.CompilerParams(needs_layout_passes=False),
    )
    out = run(x.reshape(-1), params)
    return out.reshape(B, C, H, W)


# speedup vs baseline: 3.5443x; 3.5443x over previous
import functools

import jax
import jax.numpy as jnp
from jax import lax
from jax.experimental import pallas as pl
from jax.experimental.pallas import tpu as pltpu
from jax.experimental.pallas import tpu_sc as plsc

_L = 16
_CHUNK = 24576
_UNROLL = 8


def _make_tables(params_v, iota):
    lane = lambda off: plsc.load_gather(params_v, [jnp.zeros((_L,), jnp.int32) + off])
    lb = lane(7)
    rb = lane(8)
    ls = lane(9)
    rs = lane(10)
    p0 = lane(0)
    rl = rb - lb
    inv = 1.0 / rl
    sim_left = lb - rl
    g_lo = plsc.load_gather(params_v, [jnp.maximum(iota - 1, 0)])
    fp_vec = jnp.where(iota == 0, p0 - ls * rl, g_lo)
    g_hi = plsc.load_gather(params_v, [jnp.minimum(iota, 6)])
    s_int = (g_hi - g_lo) * inv
    sl_vec = jnp.where(iota == 0, ls, jnp.where(iota >= 7, rs, s_int))
    return fp_vec, sl_vec, sim_left, inv


def _compute_chunk(src, dst, fp_ref, sl_ref, sim_left, inv):
    n_vecs = _CHUNK // _L

    @plsc.parallel_loop(0, n_vecs, 1, unroll=_UNROLL)
    def _(i):
        o = i * _L
        xv = src[pl.ds(o, _L)]
        t = (xv - sim_left) * inv
        tc = jnp.minimum(jnp.maximum(t, 0.0), 7.007)
        ri = tc.astype(jnp.int32)
        d = t - ri.astype(jnp.float32)
        fpv = plsc.load_gather(fp_ref, [ri])
        slv = plsc.load_gather(sl_ref, [ri])
        dst[pl.ds(o, _L)] = fpv + d * slv


def _pwlu_sc_body(n_slab_per_w, slab_len, n_ch, x_hbm, params_hbm, out_hbm,
                  params_v, fp_ref, sl_ref,
                  xb0, xb1, ob0, ob1, si0, si1, so0, so1):
    wid = lax.axis_index("s") * 2 + lax.axis_index("c")
    iota = lax.iota(jnp.int32, _L)
    xb, ob, si, so = [xb0, xb1], [ob0, ob1], [si0, si1], [so0, so1]

    n_chunks = slab_len // _CHUNK
    chunks = [(j, k) for j in range(n_slab_per_w) for k in range(n_chunks)]
    T = len(chunks)

    def hbm_slice(j, k):
        base = (wid * n_slab_per_w + j) * slab_len + k * _CHUNK
        return pl.ds(base, _CHUNK)

    h_in = [None] * T
    h_out = [None] * T
    h_in[0] = pltpu.async_copy(x_hbm.at[hbm_slice(0, 0)], xb[0], si[0])

    sim_left = inv = None
    for t, (j, k) in enumerate(chunks):
        if k == 0:
            c = lax.rem(wid * n_slab_per_w + j, n_ch)
            pltpu.sync_copy(params_hbm.at[pl.ds(c * _L, _L)], params_v)
            fp_vec, sl_vec, sim_left, inv = _make_tables(params_v, iota)
            fp_ref[...] = fp_vec
            sl_ref[...] = sl_vec
        if t + 1 < T:
            jn, kn = chunks[t + 1]
            h_in[t + 1] = pltpu.async_copy(
                x_hbm.at[hbm_slice(jn, kn)], xb[(t + 1) % 2], si[(t + 1) % 2])
        h_in[t].wait()
        if t >= 2:
            h_out[t - 2].wait()
        _compute_chunk(xb[t % 2], ob[t % 2], fp_ref, sl_ref, sim_left, inv)
        h_out[t] = pltpu.async_copy(
            ob[t % 2], out_hbm.at[hbm_slice(j, k)], so[t % 2])
    h_out[T - 2].wait()
    h_out[T - 1].wait()


def kernel(x, points, bounds, left_slopes, right_slopes):
    B, C, H, W = x.shape
    n = B * C * H * W
    slab_len = H * W
    n_slabs = B * C
    n_workers = 32
    assert n_slabs % n_workers == 0 and slab_len % _CHUNK == 0

    params = jnp.concatenate(
        [points, bounds, left_slopes[:, None], right_slopes[:, None],
         jnp.zeros((C, _L - 11), jnp.float32)], axis=1).reshape(-1)

    mesh = plsc.VectorSubcoreMesh(core_axis_name="c", subcore_axis_name="s")
    body = functools.partial(
        _pwlu_sc_body, n_slabs // n_workers, slab_len, C)
    run = pl.kernel(
        body,
        mesh=mesh,
        out_type=jax.ShapeDtypeStruct((n,), jnp.float32),
        scratch_types=[
            pltpu.VMEM((_L,), jnp.float32),
            pltpu.VMEM((_L,), jnp.float32),
            pltpu.VMEM((_L,), jnp.float32),
            pltpu.VMEM((_CHUNK,), jnp.float32),
            pltpu.VMEM((_CHUNK,), jnp.float32),
            pltpu.VMEM((_CHUNK,), jnp.float32),
            pltpu.VMEM((_CHUNK,), jnp.float32),
            pltpu.SemaphoreType.DMA,
            pltpu.SemaphoreType.DMA,
            pltpu.SemaphoreType.DMA,
            pltpu.SemaphoreType.DMA,
        ],
        compiler_params=pltpu.CompilerParams(needs_layout_passes=False),
    )
    out = run(x.reshape(-1), params)
    return out.reshape(B, C, H, W)


# speedup vs baseline: 7.9022x; 2.2296x over previous
import functools

import jax
import jax.numpy as jnp
from jax import lax
from jax.experimental import pallas as pl
from jax.experimental.pallas import tpu as pltpu
from jax.experimental.pallas import tpu_sc as plsc

_L = 16
_ROWS = 64


def _make_tables(params_v, iota):
    lane = lambda off: plsc.load_gather(params_v, [jnp.zeros((_L,), jnp.int32) + off])
    lb = lane(7)
    rb = lane(8)
    ls = lane(9)
    rs = lane(10)
    p0 = lane(0)
    rl = rb - lb
    inv = 1.0 / rl
    sim_left = lb - rl
    g_lo = plsc.load_gather(params_v, [jnp.maximum(iota - 1, 0)])
    fp_vec = jnp.where(iota == 0, p0 - ls * rl, g_lo)
    g_hi = plsc.load_gather(params_v, [jnp.minimum(iota, 6)])
    s_int = (g_hi - g_lo) * inv
    sl_vec = jnp.where(iota == 0, ls, jnp.where(iota >= 7, rs, s_int))
    return fp_vec, sl_vec, sim_left, inv


def _compute_chunk(src, dst, fp_ref, sl_ref, sim_ref, inv_ref, n_w):
    sim_left = sim_ref[...]
    inv = inv_ref[...]

    @plsc.parallel_loop(0, _ROWS, 1)
    def _(r):
        for v in range(n_w):
            xv = src[r, pl.ds(v * _L, _L)]
            t = (xv - sim_left) * inv
            tc = jnp.minimum(jnp.maximum(t, 0.0), 7.007)
            ri = tc.astype(jnp.int32)
            d = t - ri.astype(jnp.float32)
            fpv = plsc.load_gather(fp_ref, [ri])
            slv = plsc.load_gather(sl_ref, [ri])
            dst[r, pl.ds(v * _L, _L)] = fpv + d * slv


def _pwlu_sc_body(n_slab_per_w, n_ch, x_hbm, params_hbm, out_hbm,
                  params_v, fp_ref, sl_ref, sim_ref, inv_ref,
                  xb0, xb1, ob0, ob1, si0, si1, so0, so1):
    wid = lax.axis_index("s") * 2 + lax.axis_index("c")
    iota = lax.iota(jnp.int32, _L)

    H, W = x_hbm.shape[2], x_hbm.shape[3]
    n_w = W // _L
    n_chunks = H // _ROWS
    T = n_slab_per_w * n_chunks
    assert T % 2 == 0 and n_chunks % 2 == 0

    def x_slice(t):
        slab = wid * n_slab_per_w + lax.div(t, n_chunks)
        k = lax.rem(t, n_chunks)
        return x_hbm.at[lax.div(slab, n_ch), lax.rem(slab, n_ch),
                        pl.ds(k * _ROWS, _ROWS), :]

    def o_slice(t):
        slab = wid * n_slab_per_w + lax.div(t, n_chunks)
        k = lax.rem(t, n_chunks)
        return out_hbm.at[lax.div(slab, n_ch), lax.rem(slab, n_ch),
                          pl.ds(k * _ROWS, _ROWS), :]

    def setup_tables(t):
        slab = wid * n_slab_per_w + lax.div(t, n_chunks)
        c = lax.rem(slab, n_ch)
        pltpu.sync_copy(params_hbm.at[pl.ds(c * _L, _L)], params_v)
        fp_vec, sl_vec, sim_left, inv = _make_tables(params_v, iota)
        fp_ref[...] = fp_vec
        sl_ref[...] = sl_vec
        sim_ref[...] = sim_left
        inv_ref[...] = inv

    pltpu.async_copy(x_slice(jnp.int32(0)), xb0, si0)

    def pair_body(p, carry):
        a = 2 * p

        @pl.when(lax.rem(a, n_chunks) == 0)
        def _():
            setup_tables(a)

        pltpu.async_copy(x_slice(a + 1), xb1, si1)
        pltpu.make_async_copy(x_slice(a), xb0, si0).wait()

        @pl.when(p > 0)
        def _():
            pltpu.make_async_copy(ob0, o_slice(a - 2), so0).wait()

        _compute_chunk(xb0, ob0, fp_ref, sl_ref, sim_ref, inv_ref, n_w)
        pltpu.async_copy(ob0, o_slice(a), so0)

        @pl.when(a + 2 < T)
        def _():
            pltpu.async_copy(x_slice(a + 2), xb0, si0)

        @pl.when(p > 0)
        def _():
            pltpu.make_async_copy(ob1, o_slice(a - 1), so1).wait()

        pltpu.make_async_copy(x_slice(a + 1), xb1, si1).wait()
        _compute_chunk(xb1, ob1, fp_ref, sl_ref, sim_ref, inv_ref, n_w)
        pltpu.async_copy(ob1, o_slice(a + 1), so1)
        return carry

    lax.fori_loop(0, T // 2, pair_body, jnp.int32(0))
    pltpu.make_async_copy(ob0, o_slice(jnp.int32(T - 2)), so0).wait()
    pltpu.make_async_copy(ob1, o_slice(jnp.int32(T - 1)), so1).wait()


def kernel(x, points, bounds, left_slopes, right_slopes):
    B, C, H, W = x.shape
    n_slabs = B * C
    n_workers = 32
    assert n_slabs % n_workers == 0 and H % _ROWS == 0 and W % _L == 0

    params = jnp.concatenate(
        [points, bounds, left_slopes[:, None], right_slopes[:, None],
         jnp.zeros((C, _L - 11), jnp.float32)], axis=1).reshape(-1)

    mesh = plsc.VectorSubcoreMesh(core_axis_name="c", subcore_axis_name="s")
    body = functools.partial(_pwlu_sc_body, n_slabs // n_workers, C)
    run = pl.kernel(
        body,
        mesh=mesh,
        out_type=jax.ShapeDtypeStruct((B, C, H, W), jnp.float32),
        scratch_types=[
            pltpu.VMEM((_L,), jnp.float32),
            pltpu.VMEM((_L,), jnp.float32),
            pltpu.VMEM((_L,), jnp.float32),
            pltpu.VMEM((_L,), jnp.float32),
            pltpu.VMEM((_L,), jnp.float32),
            pltpu.VMEM((_ROWS, W), jnp.float32),
            pltpu.VMEM((_ROWS, W), jnp.float32),
            pltpu.VMEM((_ROWS, W), jnp.float32),
            pltpu.VMEM((_ROWS, W), jnp.float32),
            pltpu.SemaphoreType.DMA,
            pltpu.SemaphoreType.DMA,
            pltpu.SemaphoreType.DMA,
            pltpu.SemaphoreType.DMA,
        ],
        compiler_params=pltpu.CompilerParams(needs_layout_passes=False),
    )
    return run(x, params)


# speedup vs baseline: 8.4134x; 1.0647x over previous
import functools

import jax
import jax.numpy as jnp
from jax import lax
from jax.experimental import pallas as pl
from jax.experimental.pallas import tpu as pltpu
from jax.experimental.pallas import tpu_sc as plsc

_L = 16
_ROWS = 64


def _make_tables(params_v, iota):
    lane = lambda off: plsc.load_gather(params_v, [jnp.zeros((_L,), jnp.int32) + off])
    lb = lane(7)
    rb = lane(8)
    ls = lane(9)
    rs = lane(10)
    p0 = lane(0)
    rl = rb - lb
    inv = 1.0 / rl
    sim_left = lb - rl
    g_lo = plsc.load_gather(params_v, [jnp.maximum(iota - 1, 0)])
    fp_vec = jnp.where(iota == 0, p0 - ls * rl, g_lo)
    g_hi = plsc.load_gather(params_v, [jnp.minimum(iota, 6)])
    s_int = (g_hi - g_lo) * inv
    sl_vec = jnp.where(iota == 0, ls, jnp.where(iota >= 7, rs, s_int))
    a_vec = fp_vec - iota.astype(jnp.float32) * sl_vec
    return a_vec, sl_vec, sim_left * inv, inv


def _compute_chunk(src, dst, a_ref, sl_ref, s2_ref, inv_ref, n_w):
    s2 = s2_ref[...]
    inv = inv_ref[...]

    @plsc.parallel_loop(0, _ROWS, 1)
    def _(r):
        for v in range(n_w):
            xv = src[r, pl.ds(v * _L, _L)]
            t = xv * inv - s2
            tc = jnp.minimum(jnp.maximum(t, 0.0), 7.007)
            ri = tc.astype(jnp.int32)
            av = plsc.load_gather(a_ref, [ri])
            slv = plsc.load_gather(sl_ref, [ri])
            dst[r, pl.ds(v * _L, _L)] = av + t * slv


def _pwlu_sc_body(n_slab_per_w, n_ch, x_hbm, params_hbm, out_hbm,
                  params_v, fp_ref, sl_ref, sim_ref, inv_ref,
                  xb0, xb1, ob0, ob1, si0, si1, so0, so1):
    wid = lax.axis_index("s") * 2 + lax.axis_index("c")
    iota = lax.iota(jnp.int32, _L)

    H, W = x_hbm.shape[2], x_hbm.shape[3]
    n_w = W // _L
    n_chunks = H // _ROWS
    T = n_slab_per_w * n_chunks
    assert T % 2 == 0 and n_chunks % 2 == 0

    def x_slice(t):
        slab = wid * n_slab_per_w + lax.div(t, n_chunks)
        k = lax.rem(t, n_chunks)
        return x_hbm.at[lax.div(slab, n_ch), lax.rem(slab, n_ch),
                        pl.ds(k * _ROWS, _ROWS), :]

    def o_slice(t):
        slab = wid * n_slab_per_w + lax.div(t, n_chunks)
        k = lax.rem(t, n_chunks)
        return out_hbm.at[lax.div(slab, n_ch), lax.rem(slab, n_ch),
                          pl.ds(k * _ROWS, _ROWS), :]

    def setup_tables(t):
        slab = wid * n_slab_per_w + lax.div(t, n_chunks)
        c = lax.rem(slab, n_ch)
        pltpu.sync_copy(params_hbm.at[pl.ds(c * _L, _L)], params_v)
        fp_vec, sl_vec, sim_left, inv = _make_tables(params_v, iota)
        fp_ref[...] = fp_vec
        sl_ref[...] = sl_vec
        sim_ref[...] = sim_left
        inv_ref[...] = inv

    pltpu.async_copy(x_slice(jnp.int32(0)), xb0, si0)

    def pair_body(p, carry):
        a = 2 * p

        @pl.when(lax.rem(a, n_chunks) == 0)
        def _():
            setup_tables(a)

        pltpu.async_copy(x_slice(a + 1), xb1, si1)
        pltpu.make_async_copy(x_slice(a), xb0, si0).wait()

        @pl.when(p > 0)
        def _():
            pltpu.make_async_copy(ob0, o_slice(a - 2), so0).wait()

        _compute_chunk(xb0, ob0, fp_ref, sl_ref, sim_ref, inv_ref, n_w)
        pltpu.async_copy(ob0, o_slice(a), so0)

        @pl.when(a + 2 < T)
        def _():
            pltpu.async_copy(x_slice(a + 2), xb0, si0)

        @pl.when(p > 0)
        def _():
            pltpu.make_async_copy(ob1, o_slice(a - 1), so1).wait()

        pltpu.make_async_copy(x_slice(a + 1), xb1, si1).wait()
        _compute_chunk(xb1, ob1, fp_ref, sl_ref, sim_ref, inv_ref, n_w)
        pltpu.async_copy(ob1, o_slice(a + 1), so1)
        return carry

    lax.fori_loop(0, T // 2, pair_body, jnp.int32(0))
    pltpu.make_async_copy(ob0, o_slice(jnp.int32(T - 2)), so0).wait()
    pltpu.make_async_copy(ob1, o_slice(jnp.int32(T - 1)), so1).wait()


def kernel(x, points, bounds, left_slopes, right_slopes):
    B, C, H, W = x.shape
    n_slabs = B * C
    n_workers = 32
    assert n_slabs % n_workers == 0 and H % _ROWS == 0 and W % _L == 0

    params = jnp.concatenate(
        [points, bounds, left_slopes[:, None], right_slopes[:, None],
         jnp.zeros((C, _L - 11), jnp.float32)], axis=1).reshape(-1)

    mesh = plsc.VectorSubcoreMesh(core_axis_name="c", subcore_axis_name="s")
    body = functools.partial(_pwlu_sc_body, n_slabs // n_workers, C)
    run = pl.kernel(
        body,
        mesh=mesh,
        out_type=jax.ShapeDtypeStruct((B, C, H, W), jnp.float32),
        scratch_types=[
            pltpu.VMEM((_L,), jnp.float32),
            pltpu.VMEM((_L,), jnp.float32),
            pltpu.VMEM((_L,), jnp.float32),
            pltpu.VMEM((_L,), jnp.float32),
            pltpu.VMEM((_L,), jnp.float32),
            pltpu.VMEM((_ROWS, W), jnp.float32),
            pltpu.VMEM((_ROWS, W), jnp.float32),
            pltpu.VMEM((_ROWS, W), jnp.float32),
            pltpu.VMEM((_ROWS, W), jnp.float32),
            pltpu.SemaphoreType.DMA,
            pltpu.SemaphoreType.DMA,
            pltpu.SemaphoreType.DMA,
            pltpu.SemaphoreType.DMA,
        ],
        compiler_params=pltpu.CompilerParams(needs_layout_passes=False),
    )
    return run(x, params)
